# baseline (speedup 1.0000x reference)
import functools

import jax
import jax.numpy as jnp
import numpy as np
from jax import lax
from jax.experimental import pallas as pl
from jax.experimental.pallas import tpu as pltpu
from jax.experimental.pallas import tpu_sc as plsc

SCALE_MULTI = 0.5
LOG2_T = 16
BASE_RES = 16
PLS = 2
L = 10
DIM = 4
HIDDEN = 64
DENSITY_OFFSET = -4.0

_P2 = int(np.uint32(2654435761).view(np.int32))
_P3 = int(np.uint32(805459861).view(np.int32))

_SELU_ALPHA = 1.6732632423543772
_SELU_SCALE = 1.0507009873554805

NW = 32
C = 64


def _levels():
    T = 2 ** LOG2_T
    out = []
    off = 0
    for l in range(L):
        res = int(np.floor(BASE_RES * (PLS ** l)))
        size = min((res + 1) ** 3, T)
        out.append((res, size, off, size == (res + 1) ** 3))
        off += size
    return out

_LEVELS = _levels()
_TOTAL = sum(s for _, s, _, _ in _LEVELS)


def _enc_body(t01, t23, xx, xy, xz, out,
              s01, s23, xcb, idxb, frb, r01, r23, accb, sem, semx):
    npts = out.shape[1] // NW
    nch = npts // C
    cid = lax.axis_index("c")
    sid = lax.axis_index("s")
    wid = sid * 2 + cid
    base = wid * npts

    total = t01.shape[0]
    share = total // 16
    pltpu.sync_copy(t01.at[pl.ds(sid * share, share)], s01.at[pl.ds(sid * share, share)])
    pltpu.sync_copy(t23.at[pl.ds(sid * share, share)], s23.at[pl.ds(sid * share, share)])
    plsc.subcore_barrier()

    def xc_copies(g, buf):
        p0 = g * C
        return [pltpu.make_async_copy(xx.at[pl.ds(base + p0, C)], xcb.at[buf, 0], semx),
                pltpu.make_async_copy(xy.at[pl.ds(base + p0, C)], xcb.at[buf, 1], semx),
                pltpu.make_async_copy(xz.at[pl.ds(base + p0, C)], xcb.at[buf, 2], semx)]

    def a_and_fire(k):
        kb = k & 1
        for cp in xc_copies(k, kb):
            cp.wait()

        @pl.when(k + 1 < nch)
        def _prefetch():
            for cp in xc_copies(k + 1, 1 - kb):
                cp.start()

        def phase_a(i, c2):
            off = i * 16
            vx = xcb[kb, 0, pl.ds(off, 16)]
            vy = xcb[kb, 1, pl.ds(off, 16)]
            vz = xcb[kb, 2, pl.ds(off, 16)]
            for l, (res, size, loff, dense) in enumerate(_LEVELS):
                px = vx * float(res)
                py = vy * float(res)
                pz = vz * float(res)
                ix = px.astype(jnp.int32)
                iy = py.astype(jnp.int32)
                iz = pz.astype(jnp.int32)
                frb[kb, pl.ds(l * C + i * 16, 16)] = px - ix.astype(jnp.float32)
                frb[kb, pl.ds((L + l) * C + i * 16, 16)] = py - iy.astype(jnp.float32)
                frb[kb, pl.ds((2 * L + l) * C + i * 16, 16)] = pz - iz.astype(jnp.float32)
                if dense:
                    r1 = res + 1
                    ax = (ix, ix + 1)
                    ay = (iy * r1, iy * r1 + r1)
                    az = (iz * (r1 * r1) + loff, iz * (r1 * r1) + (r1 * r1) + loff)
                    for c in range(8):
                        v = ax[c & 1] + ay[(c >> 1) & 1] + az[(c >> 2) & 1]
                        idxb[kb, l * 8 + c, pl.ds(i * 16, 16)] = v
                else:
                    hx = (ix, ix + 1)
                    hy = (iy * _P2, iy * _P2 + _P2)
                    hz = (iz * _P3, iz * _P3 + _P3)
                    for c in range(8):
                        h = hx[c & 1] ^ hy[(c >> 1) & 1] ^ hz[(c >> 2) & 1]
                        idxb[kb, l * 8 + c, pl.ds(i * 16, 16)] = (h & 0xFFFF) + loff
            return c2

        lax.fori_loop(0, C // 16, phase_a, 0)

        for j in range(L * 8):
            pltpu.make_async_copy(s01.at[idxb.at[kb, j]], r01.at[kb, j], sem).start()
            pltpu.make_async_copy(s23.at[idxb.at[kb, j]], r23.at[kb, j], sem).start()

    for cp in xc_copies(0, 0):
        cp.start()
    a_and_fire(0)

    def chunk_body(g, carry):
        p0 = g * C
        gb = g & 1

        @pl.when(g + 1 < nch)
        def _next():
            a_and_fire(g + 1)

        for j in range(L * 8):
            pltpu.make_async_copy(s01.at[idxb.at[gb, j]], r01.at[gb, j], sem).wait()
            pltpu.make_async_copy(s23.at[idxb.at[gb, j]], r23.at[gb, j], sem).wait()

        himask = jnp.int32(-65536)

        def phase_c(i, c2):
            o16 = i * 16
            for l in range(L):
                fx = frb[gb, pl.ds(l * C + o16, 16)]
                fy = frb[gb, pl.ds((L + l) * C + o16, 16)]
                fz = frb[gb, pl.ds((2 * L + l) * C + o16, 16)]
                m = []
                for cyz in range(4):
                    w01a = r01[gb, l * 8 + 2 * cyz, pl.ds(o16, 16)]
                    w23a = r23[gb, l * 8 + 2 * cyz, pl.ds(o16, 16)]
                    w01b = r01[gb, l * 8 + 2 * cyz + 1, pl.ds(o16, 16)]
                    w23b = r23[gb, l * 8 + 2 * cyz + 1, pl.ds(o16, 16)]
                    da = (lax.bitcast_convert_type(w01a & himask, jnp.float32),
                          lax.bitcast_convert_type(w01a << 16, jnp.float32),
                          lax.bitcast_convert_type(w23a & himask, jnp.float32),
                          lax.bitcast_convert_type(w23a << 16, jnp.float32))
                    db = (lax.bitcast_convert_type(w01b & himask, jnp.float32),
                          lax.bitcast_convert_type(w01b << 16, jnp.float32),
                          lax.bitcast_convert_type(w23b & himask, jnp.float32),
                          lax.bitcast_convert_type(w23b << 16, jnp.float32))
                    m.append(tuple(da[d] + (db[d] - da[d]) * fx for d in range(4)))
                n0 = tuple(m[0][d] + (m[1][d] - m[0][d]) * fy for d in range(4))
                n1 = tuple(m[2][d] + (m[3][d] - m[2][d]) * fy for d in range(4))
                for d in range(4):
                    accb[l * 4 + d, pl.ds(o16, 16)] = n0[d] + (n1[d] - n0[d]) * fz
            return c2

        lax.fori_loop(0, C // 16, phase_c, 0)

        pltpu.sync_copy(accb, out.at[:, pl.ds(base + p0, C)])
        return carry

    lax.fori_loop(0, nch, chunk_body, 0)


def _sc_encode(x, t01, t23):
    n = x.shape[0]
    npts = n // NW
    total = t01.shape[0]
    mesh = plsc.VectorSubcoreMesh(core_axis_name="c", subcore_axis_name="s")
    f = pl.kernel(
        _enc_body,
        out_type=jax.ShapeDtypeStruct((L * DIM, n), jnp.float32),
        mesh=mesh,
        compiler_params=pltpu.CompilerParams(
            use_tc_tiling_on_sc=False, needs_layout_passes=False),
        scratch_types=[
            pltpu.VMEM_SHARED((total,), jnp.int32),
            pltpu.VMEM_SHARED((total,), jnp.int32),
            pltpu.VMEM((2, 3, C), jnp.float32),
            pltpu.VMEM((2, L * 8, C), jnp.int32),
            pltpu.VMEM((2, 3 * L * C), jnp.float32),
            pltpu.VMEM((2, L * 8, C), jnp.int32),
            pltpu.VMEM((2, L * 8, C), jnp.int32),
            pltpu.VMEM((L * DIM, C), jnp.float32),
            pltpu.SemaphoreType.DMA,
            pltpu.SemaphoreType.DMA,
        ],
    )
    return f(t01, t23, x[:, 0], x[:, 1], x[:, 2])


def _pack_table(table):
    tb = lax.bitcast_convert_type(table.astype(jnp.bfloat16), jnp.uint16).astype(jnp.uint32)
    w01 = (tb[:, 0] << 16) | tb[:, 1]
    w23 = (tb[:, 2] << 16) | tb[:, 3]
    pad = (-table.shape[0]) % (16 * 8)
    w01 = jnp.pad(w01, (0, pad))
    w23 = jnp.pad(w23, (0, pad))
    return w01.astype(jnp.int32), w23.astype(jnp.int32)


def _f16_round(v):
    b = jax.lax.bitcast_convert_type(v, jnp.int32)
    b = (b + jnp.int32(0xFFF) + ((b >> 13) & 1)) & jnp.int32(-8192)
    return jax.lax.bitcast_convert_type(b, jnp.float32)


def _selu(v):
    return _SELU_SCALE * jnp.where(v > 0, v, _SELU_ALPHA * (jnp.exp(v) - 1.0))


def _head_body(enc_ref, cr_ref, w1_ref, b1_ref, w2_ref, b2_ref, w3_ref, b3_ref, out_ref):
    enc = enc_ref[...]
    cr = cr_ref[...]
    cr_h = _f16_round(cr) * SCALE_MULTI
    n_mod = (jnp.arange(L * DIM, dtype=jnp.int32) % L).reshape(L * DIM, 1).astype(jnp.float32)
    denom = jnp.sqrt(jnp.maximum(PLS * 4.0 * n_mod * cr_h, 1e-8))
    scaling = _f16_round(jax.lax.erf(1.0 / jnp.maximum(denom, 1e-8)))
    h = enc * scaling
    h1 = _selu(jnp.dot(w1_ref[...], h, preferred_element_type=jnp.float32) + b1_ref[...])
    h2 = _selu(jnp.dot(w2_ref[...], h1, preferred_element_type=jnp.float32) + b2_ref[...])
    sigma = jnp.dot(w3_ref[...], h2, preferred_element_type=jnp.float32) + b3_ref[...]
    out_ref[...] = jnp.exp(jnp.clip(sigma + DENSITY_OFFSET, -15.0, 15.0))


def _mlp_head(enc_t, cr, W1, b1, W2, b2, W3, b3, blk=2048):
    n = enc_t.shape[1]
    grid = n // blk
    full = lambda *shape: pl.BlockSpec(shape, lambda i: (0,) * len(shape))
    out = pl.pallas_call(
        _head_body,
        grid=(grid,),
        in_specs=[
            pl.BlockSpec((L * DIM, blk), lambda i: (0, i)),
            pl.BlockSpec((1, blk), lambda i: (0, i)),
            full(HIDDEN, L * DIM),
            full(HIDDEN, 1),
            full(HIDDEN, HIDDEN),
            full(HIDDEN, 1),
            full(1, HIDDEN),
            full(1, 1),
        ],
        out_specs=pl.BlockSpec((1, blk), lambda i: (0, i)),
        out_shape=jax.ShapeDtypeStruct((1, n), jnp.float32),
    )(enc_t, cr.reshape(1, n), W1.T, b1.reshape(HIDDEN, 1),
      W2.T, b2.reshape(HIDDEN, 1), W3.T, b3.reshape(1, 1))
    return out.reshape(n, 1)


def kernel(x, cr, table, W1, b1, W2, b2, W3, b3):
    t01, t23 = _pack_table(table)
    enc_t = _sc_encode(x, t01, t23)
    return _mlp_head(enc_t, cr, W1, b1, W2, b2, W3, b3)


# speedup vs baseline: 1.1993x; 1.1993x over previous
import functools

import jax
import jax.numpy as jnp
import numpy as np
from jax import lax
from jax.experimental import pallas as pl
from jax.experimental.pallas import tpu as pltpu
from jax.experimental.pallas import tpu_sc as plsc

SCALE_MULTI = 0.5
LOG2_T = 16
BASE_RES = 16
PLS = 2
L = 10
DIM = 4
HIDDEN = 64
DENSITY_OFFSET = -4.0

_P2 = int(np.uint32(2654435761).view(np.int32))
_P3 = int(np.uint32(805459861).view(np.int32))

_SELU_ALPHA = 1.6732632423543772
_SELU_SCALE = 1.0507009873554805

NW = 32
C = 128


def _levels():
    T = 2 ** LOG2_T
    out = []
    off = 0
    for l in range(L):
        res = int(np.floor(BASE_RES * (PLS ** l)))
        size = min((res + 1) ** 3, T)
        out.append((res, size, off, size == (res + 1) ** 3))
        off += size
    return out

_LEVELS = _levels()
_TOTAL = sum(s for _, s, _, _ in _LEVELS)


def _enc_body(tpk, xx, xy, xz, out,
              spk, xcb, idxb, frb, rpk, accb, sem, semx):
    npts = out.shape[1] // NW
    nch = npts // C
    cid = lax.axis_index("c")
    sid = lax.axis_index("s")
    wid = sid * 2 + cid
    base = wid * npts

    total = tpk.shape[0]
    share = total // 16
    pltpu.sync_copy(tpk.at[pl.ds(sid * share, share)], spk.at[pl.ds(sid * share, share)])
    plsc.subcore_barrier()

    def xc_copies(g, buf):
        p0 = g * C
        return [pltpu.make_async_copy(xx.at[pl.ds(base + p0, C)], xcb.at[buf, 0], semx),
                pltpu.make_async_copy(xy.at[pl.ds(base + p0, C)], xcb.at[buf, 1], semx),
                pltpu.make_async_copy(xz.at[pl.ds(base + p0, C)], xcb.at[buf, 2], semx)]

    for cp in xc_copies(0, 0):
        cp.start()

    def chunk_body(g, carry):
        p0 = g * C
        gb = g & 1

        for cp in xc_copies(g, gb):
            cp.wait()

        @pl.when(g + 1 < nch)
        def _prefetch():
            for cp in xc_copies(g + 1, 1 - gb):
                cp.start()

        def phase_a(i, c2):
            off = i * 16
            vx = xcb[gb, 0, pl.ds(off, 16)]
            vy = xcb[gb, 1, pl.ds(off, 16)]
            vz = xcb[gb, 2, pl.ds(off, 16)]
            for l, (res, size, loff, dense) in enumerate(_LEVELS):
                px = vx * float(res)
                py = vy * float(res)
                pz = vz * float(res)
                ix = px.astype(jnp.int32)
                iy = py.astype(jnp.int32)
                iz = pz.astype(jnp.int32)
                frb[pl.ds(l * C + i * 16, 16)] = px - ix.astype(jnp.float32)
                frb[pl.ds((L + l) * C + i * 16, 16)] = py - iy.astype(jnp.float32)
                frb[pl.ds((2 * L + l) * C + i * 16, 16)] = pz - iz.astype(jnp.float32)
                if dense:
                    r1 = res + 1
                    ax = (ix, ix + 1)
                    ay = (iy * r1, iy * r1 + r1)
                    az = (iz * (r1 * r1) + loff, iz * (r1 * r1) + (r1 * r1) + loff)
                    for c in range(8):
                        v = ax[c & 1] + ay[(c >> 1) & 1] + az[(c >> 2) & 1]
                        idxb[l * 8 + c, pl.ds(i * 16, 16)] = v
                else:
                    hx = (ix, ix + 1)
                    hy = (iy * _P2, iy * _P2 + _P2)
                    hz = (iz * _P3, iz * _P3 + _P3)
                    for c in range(8):
                        h = hx[c & 1] ^ hy[(c >> 1) & 1] ^ hz[(c >> 2) & 1]
                        idxb[l * 8 + c, pl.ds(i * 16, 16)] = (h & 0xFFFF) + loff
            return c2

        lax.fori_loop(0, C // 16, phase_a, 0)

        copies = []
        for j in range(L * 8):
            cp = pltpu.make_async_copy(spk.at[idxb.at[j]], rpk.at[j], sem)
            cp.start()
            copies.append(cp)
        for cp in copies:
            cp.wait()

        kfix = jnp.float32(2.0 ** 99)

        def _dec(w):
            outs = []
            for dd in range(4):
                t = w >> (24 - 8 * dd) if dd < 3 else w
                b = ((t & 0x7F) << 21) | ((t & 0x80) << 24)
                outs.append(lax.bitcast_convert_type(b, jnp.float32))
            return outs

        def phase_c(i, c2):
            o16 = i * 16
            for l in range(L):
                fx = frb[pl.ds(l * C + o16, 16)]
                fy = frb[pl.ds((L + l) * C + o16, 16)]
                fz = frb[pl.ds((2 * L + l) * C + o16, 16)]
                m = []
                for cyz in range(4):
                    da = _dec(rpk[l * 8 + 2 * cyz, pl.ds(o16, 16)])
                    db = _dec(rpk[l * 8 + 2 * cyz + 1, pl.ds(o16, 16)])
                    m.append(tuple(da[d] + (db[d] - da[d]) * fx for d in range(4)))
                n0 = tuple(m[0][d] + (m[1][d] - m[0][d]) * fy for d in range(4))
                n1 = tuple(m[2][d] + (m[3][d] - m[2][d]) * fy for d in range(4))
                for d in range(4):
                    accb[l * 4 + d, pl.ds(o16, 16)] = (n0[d] + (n1[d] - n0[d]) * fz) * kfix
            return c2

        lax.fori_loop(0, C // 16, phase_c, 0)

        pltpu.sync_copy(accb, out.at[:, pl.ds(base + p0, C)])
        return carry

    lax.fori_loop(0, nch, chunk_body, 0)


def _sc_encode(x, tpk):
    n = x.shape[0]
    npts = n // NW
    total = tpk.shape[0]
    mesh = plsc.VectorSubcoreMesh(core_axis_name="c", subcore_axis_name="s")
    f = pl.kernel(
        _enc_body,
        out_type=jax.ShapeDtypeStruct((L * DIM, n), jnp.float32),
        mesh=mesh,
        compiler_params=pltpu.CompilerParams(
            use_tc_tiling_on_sc=False, needs_layout_passes=False),
        scratch_types=[
            pltpu.VMEM_SHARED((total,), jnp.int32),
            pltpu.VMEM((2, 3, C), jnp.float32),
            pltpu.VMEM((L * 8, C), jnp.int32),
            pltpu.VMEM((3 * L * C,), jnp.float32),
            pltpu.VMEM((L * 8, C), jnp.int32),
            pltpu.VMEM((L * DIM, C), jnp.float32),
            pltpu.SemaphoreType.DMA,
            pltpu.SemaphoreType.DMA,
        ],
    )
    return f(tpk, x[:, 0], x[:, 1], x[:, 2])


def _pack_table(table):
    tb = lax.bitcast_convert_type((table * 8192.0).astype(jnp.float8_e5m2),
                                  jnp.uint8).astype(jnp.uint32)
    w = (tb[:, 0] << 24) | (tb[:, 1] << 16) | (tb[:, 2] << 8) | tb[:, 3]
    pad = (-table.shape[0]) % (16 * 8)
    return jnp.pad(w, (0, pad)).astype(jnp.int32)


def _f16_round(v):
    b = jax.lax.bitcast_convert_type(v, jnp.int32)
    b = (b + jnp.int32(0xFFF) + ((b >> 13) & 1)) & jnp.int32(-8192)
    return jax.lax.bitcast_convert_type(b, jnp.float32)


def _selu(v):
    return _SELU_SCALE * jnp.where(v > 0, v, _SELU_ALPHA * (jnp.exp(v) - 1.0))


def _head_body(enc_ref, cr_ref, w1_ref, b1_ref, w2_ref, b2_ref, w3_ref, b3_ref, out_ref):
    enc = enc_ref[...]
    cr = cr_ref[...]
    cr_h = _f16_round(cr) * SCALE_MULTI
    n_mod = (jnp.arange(L * DIM, dtype=jnp.int32) % L).reshape(L * DIM, 1).astype(jnp.float32)
    denom = jnp.sqrt(jnp.maximum(PLS * 4.0 * n_mod * cr_h, 1e-8))
    scaling = _f16_round(jax.lax.erf(1.0 / jnp.maximum(denom, 1e-8)))
    h = enc * scaling
    h1 = _selu(jnp.dot(w1_ref[...], h, preferred_element_type=jnp.float32) + b1_ref[...])
    h2 = _selu(jnp.dot(w2_ref[...], h1, preferred_element_type=jnp.float32) + b2_ref[...])
    sigma = jnp.dot(w3_ref[...], h2, preferred_element_type=jnp.float32) + b3_ref[...]
    out_ref[...] = jnp.exp(jnp.clip(sigma + DENSITY_OFFSET, -15.0, 15.0))


def _mlp_head(enc_t, cr, W1, b1, W2, b2, W3, b3, blk=2048):
    n = enc_t.shape[1]
    grid = n // blk
    full = lambda *shape: pl.BlockSpec(shape, lambda i: (0,) * len(shape))
    out = pl.pallas_call(
        _head_body,
        grid=(grid,),
        in_specs=[
            pl.BlockSpec((L * DIM, blk), lambda i: (0, i)),
            pl.BlockSpec((1, blk), lambda i: (0, i)),
            full(HIDDEN, L * DIM),
            full(HIDDEN, 1),
            full(HIDDEN, HIDDEN),
            full(HIDDEN, 1),
            full(1, HIDDEN),
            full(1, 1),
        ],
        out_specs=pl.BlockSpec((1, blk), lambda i: (0, i)),
        out_shape=jax.ShapeDtypeStruct((1, n), jnp.float32),
    )(enc_t, cr.reshape(1, n), W1.T, b1.reshape(HIDDEN, 1),
      W2.T, b2.reshape(HIDDEN, 1), W3.T, b3.reshape(1, 1))
    return out.reshape(n, 1)


def kernel(x, cr, table, W1, b1, W2, b2, W3, b3):
    tpk = _pack_table(table)
    enc_t = _sc_encode(x, tpk)
    return _mlp_head(enc_t, cr, W1, b1, W2, b2, W3, b3)


# speedup vs baseline: 1.2367x; 1.0312x over previous
import functools

import jax
import jax.numpy as jnp
import numpy as np
from jax import lax
from jax.experimental import pallas as pl
from jax.experimental.pallas import tpu as pltpu
from jax.experimental.pallas import tpu_sc as plsc

SCALE_MULTI = 0.5
LOG2_T = 16
BASE_RES = 16
PLS = 2
L = 10
DIM = 4
HIDDEN = 64
DENSITY_OFFSET = -4.0

_P2 = int(np.uint32(2654435761).view(np.int32))
_P3 = int(np.uint32(805459861).view(np.int32))

_SELU_ALPHA = 1.6732632423543772
_SELU_SCALE = 1.0507009873554805

NW = 32
C = 128


def _levels():
    T = 2 ** LOG2_T
    out = []
    off = 0
    for l in range(L):
        res = int(np.floor(BASE_RES * (PLS ** l)))
        size = min((res + 1) ** 3, T)
        out.append((res, size, off, size == (res + 1) ** 3))
        off += size
    return out

_LEVELS = _levels()
_TOTAL = sum(s for _, s, _, _ in _LEVELS)


def _enc_body(tpk, xx, xy, xz, out,
              spk, xcb, idxb, frb, rpk, accb, sem, semx):
    npts = out.shape[1] // NW
    nch = npts // C
    cid = lax.axis_index("c")
    sid = lax.axis_index("s")
    wid = sid * 2 + cid
    base = wid * npts

    total = tpk.shape[0]
    share = total // 16
    pltpu.sync_copy(tpk.at[pl.ds(sid * share, share)], spk.at[pl.ds(sid * share, share)])
    plsc.subcore_barrier()

    def xc_copies(g, buf):
        p0 = g * C
        return [pltpu.make_async_copy(xx.at[pl.ds(base + p0, C)], xcb.at[buf, 0], semx),
                pltpu.make_async_copy(xy.at[pl.ds(base + p0, C)], xcb.at[buf, 1], semx),
                pltpu.make_async_copy(xz.at[pl.ds(base + p0, C)], xcb.at[buf, 2], semx)]

    def a_and_fire(k):
        kb = k & 1
        for cp in xc_copies(k, kb):
            cp.wait()

        @pl.when(k + 1 < nch)
        def _prefetch():
            for cp in xc_copies(k + 1, 1 - kb):
                cp.start()

        def phase_a(i, c2):
            off = i * 16
            vx = xcb[kb, 0, pl.ds(off, 16)]
            vy = xcb[kb, 1, pl.ds(off, 16)]
            vz = xcb[kb, 2, pl.ds(off, 16)]
            for l, (res, size, loff, dense) in enumerate(_LEVELS):
                px = vx * float(res)
                py = vy * float(res)
                pz = vz * float(res)
                ix = px.astype(jnp.int32)
                iy = py.astype(jnp.int32)
                iz = pz.astype(jnp.int32)
                frb[kb, pl.ds(l * C + i * 16, 16)] = px - ix.astype(jnp.float32)
                frb[kb, pl.ds((L + l) * C + i * 16, 16)] = py - iy.astype(jnp.float32)
                frb[kb, pl.ds((2 * L + l) * C + i * 16, 16)] = pz - iz.astype(jnp.float32)
                if dense:
                    r1 = res + 1
                    ax = (ix, ix + 1)
                    ay = (iy * r1, iy * r1 + r1)
                    az = (iz * (r1 * r1) + loff, iz * (r1 * r1) + (r1 * r1) + loff)
                    for c in range(8):
                        v = ax[c & 1] + ay[(c >> 1) & 1] + az[(c >> 2) & 1]
                        idxb[kb, l * 8 + c, pl.ds(i * 16, 16)] = v
                else:
                    hx = (ix, ix + 1)
                    hy = (iy * _P2, iy * _P2 + _P2)
                    hz = (iz * _P3, iz * _P3 + _P3)
                    for c in range(8):
                        h = hx[c & 1] ^ hy[(c >> 1) & 1] ^ hz[(c >> 2) & 1]
                        idxb[kb, l * 8 + c, pl.ds(i * 16, 16)] = (h & 0xFFFF) + loff
            return c2

        lax.fori_loop(0, C // 16, phase_a, 0)

        for j in range(L * 8):
            pltpu.make_async_copy(spk.at[idxb.at[kb, j]], rpk.at[kb, j], sem).start()

    for cp in xc_copies(0, 0):
        cp.start()
    a_and_fire(0)

    def chunk_body(g, carry):
        p0 = g * C
        gb = g & 1

        @pl.when(g + 1 < nch)
        def _next():
            a_and_fire(g + 1)

        for j in range(L * 8):
            pltpu.make_async_copy(spk.at[idxb.at[gb, j]], rpk.at[gb, j], sem).wait()

        kfix = jnp.float32(2.0 ** 99)

        def _dec(w):
            outs = []
            for dd in range(4):
                t = w >> (24 - 8 * dd) if dd < 3 else w
                b = ((t & 0x7F) << 21) | ((t & 0x80) << 24)
                outs.append(lax.bitcast_convert_type(b, jnp.float32))
            return outs

        def phase_c(i, c2):
            o16 = i * 16
            for l in range(L):
                fx = frb[gb, pl.ds(l * C + o16, 16)]
                fy = frb[gb, pl.ds((L + l) * C + o16, 16)]
                fz = frb[gb, pl.ds((2 * L + l) * C + o16, 16)]
                m = []
                for cyz in range(4):
                    da = _dec(rpk[gb, l * 8 + 2 * cyz, pl.ds(o16, 16)])
                    db = _dec(rpk[gb, l * 8 + 2 * cyz + 1, pl.ds(o16, 16)])
                    m.append(tuple(da[d] + (db[d] - da[d]) * fx for d in range(4)))
                n0 = tuple(m[0][d] + (m[1][d] - m[0][d]) * fy for d in range(4))
                n1 = tuple(m[2][d] + (m[3][d] - m[2][d]) * fy for d in range(4))
                for d in range(4):
                    accb[l * 4 + d, pl.ds(o16, 16)] = (n0[d] + (n1[d] - n0[d]) * fz) * kfix
            return c2

        lax.fori_loop(0, C // 16, phase_c, 0)

        pltpu.sync_copy(accb, out.at[:, pl.ds(base + p0, C)])
        return carry

    lax.fori_loop(0, nch, chunk_body, 0)


def _sc_encode(x, tpk):
    n = x.shape[0]
    npts = n // NW
    total = tpk.shape[0]
    mesh = plsc.VectorSubcoreMesh(core_axis_name="c", subcore_axis_name="s")
    f = pl.kernel(
        _enc_body,
        out_type=jax.ShapeDtypeStruct((L * DIM, n), jnp.float32),
        mesh=mesh,
        compiler_params=pltpu.CompilerParams(
            use_tc_tiling_on_sc=False, needs_layout_passes=False),
        scratch_types=[
            pltpu.VMEM_SHARED((total,), jnp.int32),
            pltpu.VMEM((2, 3, C), jnp.float32),
            pltpu.VMEM((2, L * 8, C), jnp.int32),
            pltpu.VMEM((2, 3 * L * C), jnp.float32),
            pltpu.VMEM((2, L * 8, C), jnp.int32),
            pltpu.VMEM((L * DIM, C), jnp.float32),
            pltpu.SemaphoreType.DMA,
            pltpu.SemaphoreType.DMA,
        ],
    )
    return f(tpk, x[:, 0], x[:, 1], x[:, 2])


def _pack_table(table):
    tb = lax.bitcast_convert_type((table * 8192.0).astype(jnp.float8_e5m2),
                                  jnp.uint8).astype(jnp.uint32)
    w = (tb[:, 0] << 24) | (tb[:, 1] << 16) | (tb[:, 2] << 8) | tb[:, 3]
    pad = (-table.shape[0]) % (16 * 8)
    return jnp.pad(w, (0, pad)).astype(jnp.int32)


def _f16_round(v):
    b = jax.lax.bitcast_convert_type(v, jnp.int32)
    b = (b + jnp.int32(0xFFF) + ((b >> 13) & 1)) & jnp.int32(-8192)
    return jax.lax.bitcast_convert_type(b, jnp.float32)


def _selu(v):
    return _SELU_SCALE * jnp.where(v > 0, v, _SELU_ALPHA * (jnp.exp(v) - 1.0))


def _head_body(enc_ref, cr_ref, w1_ref, b1_ref, w2_ref, b2_ref, w3_ref, b3_ref, out_ref):
    enc = enc_ref[...]
    cr = cr_ref[...]
    cr_h = _f16_round(cr) * SCALE_MULTI
    n_mod = (jnp.arange(L * DIM, dtype=jnp.int32) % L).reshape(L * DIM, 1).astype(jnp.float32)
    denom = jnp.sqrt(jnp.maximum(PLS * 4.0 * n_mod * cr_h, 1e-8))
    scaling = _f16_round(jax.lax.erf(1.0 / jnp.maximum(denom, 1e-8)))
    h = enc * scaling
    h1 = _selu(jnp.dot(w1_ref[...], h, preferred_element_type=jnp.float32) + b1_ref[...])
    h2 = _selu(jnp.dot(w2_ref[...], h1, preferred_element_type=jnp.float32) + b2_ref[...])
    sigma = jnp.dot(w3_ref[...], h2, preferred_element_type=jnp.float32) + b3_ref[...]
    out_ref[...] = jnp.exp(jnp.clip(sigma + DENSITY_OFFSET, -15.0, 15.0))


def _mlp_head(enc_t, cr, W1, b1, W2, b2, W3, b3, blk=2048):
    n = enc_t.shape[1]
    grid = n // blk
    full = lambda *shape: pl.BlockSpec(shape, lambda i: (0,) * len(shape))
    out = pl.pallas_call(
        _head_body,
        grid=(grid,),
        in_specs=[
            pl.BlockSpec((L * DIM, blk), lambda i: (0, i)),
            pl.BlockSpec((1, blk), lambda i: (0, i)),
            full(HIDDEN, L * DIM),
            full(HIDDEN, 1),
            full(HIDDEN, HIDDEN),
            full(HIDDEN, 1),
            full(1, HIDDEN),
            full(1, 1),
        ],
        out_specs=pl.BlockSpec((1, blk), lambda i: (0, i)),
        out_shape=jax.ShapeDtypeStruct((1, n), jnp.float32),
    )(enc_t, cr.reshape(1, n), W1.T, b1.reshape(HIDDEN, 1),
      W2.T, b2.reshape(HIDDEN, 1), W3.T, b3.reshape(1, 1))
    return out.reshape(n, 1)


def kernel(x, cr, table, W1, b1, W2, b2, W3, b3):
    tpk = _pack_table(table)
    enc_t = _sc_encode(x, tpk)
    return _mlp_head(enc_t, cr, W1, b1, W2, b2, W3, b3)


# speedup vs baseline: 1.2370x; 1.0003x over previous
import functools

import jax
import jax.numpy as jnp
import numpy as np
from jax import lax
from jax.experimental import pallas as pl
from jax.experimental.pallas import tpu as pltpu
from jax.experimental.pallas import tpu_sc as plsc

SCALE_MULTI = 0.5
LOG2_T = 16
BASE_RES = 16
PLS = 2
L = 10
DIM = 4
HIDDEN = 64
DENSITY_OFFSET = -4.0

_P2 = int(np.uint32(2654435761).view(np.int32))
_P3 = int(np.uint32(805459861).view(np.int32))

_SELU_ALPHA = 1.6732632423543772
_SELU_SCALE = 1.0507009873554805

NW = 32
C = 128


def _levels():
    T = 2 ** LOG2_T
    out = []
    off = 0
    for l in range(L):
        res = int(np.floor(BASE_RES * (PLS ** l)))
        size = min((res + 1) ** 3, T)
        out.append((res, size, off, size == (res + 1) ** 3))
        off += size
    return out

_LEVELS = _levels()
_TOTAL = sum(s for _, s, _, _ in _LEVELS)


def _enc_body(tpk, xx, xy, xz, out,
              spk, xcb, idxb, frb, rpk, accb, sem, semx):
    npts = out.shape[1] // NW
    nch = npts // C
    cid = lax.axis_index("c")
    sid = lax.axis_index("s")
    wid = sid * 2 + cid
    base = wid * npts

    total = tpk.shape[0]
    share = total // 16
    pltpu.sync_copy(tpk.at[pl.ds(sid * share, share)], spk.at[pl.ds(sid * share, share)])
    plsc.subcore_barrier()

    def xc_copies(g, buf):
        p0 = g * C
        return [pltpu.make_async_copy(xx.at[pl.ds(base + p0, C)], xcb.at[buf, 0], semx),
                pltpu.make_async_copy(xy.at[pl.ds(base + p0, C)], xcb.at[buf, 1], semx),
                pltpu.make_async_copy(xz.at[pl.ds(base + p0, C)], xcb.at[buf, 2], semx)]

    def a_and_fire(k):
        kb = k & 1
        for cp in xc_copies(k, kb):
            cp.wait()

        @pl.when(k + 1 < nch)
        def _prefetch():
            for cp in xc_copies(k + 1, 1 - kb):
                cp.start()

        def phase_a(i, c2):
            off = i * 16
            vx = xcb[kb, 0, pl.ds(off, 16)]
            vy = xcb[kb, 1, pl.ds(off, 16)]
            vz = xcb[kb, 2, pl.ds(off, 16)]
            for l, (res, size, loff, dense) in enumerate(_LEVELS):
                px = vx * float(res)
                py = vy * float(res)
                pz = vz * float(res)
                ix = px.astype(jnp.int32)
                iy = py.astype(jnp.int32)
                iz = pz.astype(jnp.int32)
                frb[kb, pl.ds(l * C + i * 16, 16)] = px - ix.astype(jnp.float32)
                frb[kb, pl.ds((L + l) * C + i * 16, 16)] = py - iy.astype(jnp.float32)
                frb[kb, pl.ds((2 * L + l) * C + i * 16, 16)] = pz - iz.astype(jnp.float32)
                if dense:
                    r1 = res + 1
                    ax = (ix, ix + 1)
                    ay = (iy * r1, iy * r1 + r1)
                    az = (iz * (r1 * r1) + loff, iz * (r1 * r1) + (r1 * r1) + loff)
                    for c in range(8):
                        v = ax[c & 1] + ay[(c >> 1) & 1] + az[(c >> 2) & 1]
                        idxb[kb, l * 8 + c, pl.ds(i * 16, 16)] = v
                else:
                    hx = (ix, ix + 1)
                    hy = (iy * _P2, iy * _P2 + _P2)
                    hz = (iz * _P3, iz * _P3 + _P3)
                    for c in range(8):
                        h = hx[c & 1] ^ hy[(c >> 1) & 1] ^ hz[(c >> 2) & 1]
                        idxb[kb, l * 8 + c, pl.ds(i * 16, 16)] = (h & 0xFFFF) + loff
            return c2

        lax.fori_loop(0, C // 16, phase_a, 0)

        for j in range(L * 8):
            pltpu.make_async_copy(spk.at[idxb.at[kb, j]], rpk.at[kb, j], sem).start()

    for cp in xc_copies(0, 0):
        cp.start()
    a_and_fire(0)

    def chunk_body(g, carry):
        p0 = g * C
        gb = g & 1

        @pl.when(g + 1 < nch)
        def _next():
            a_and_fire(g + 1)

        for j in range(L * 8):
            pltpu.make_async_copy(spk.at[idxb.at[gb, j]], rpk.at[gb, j], sem).wait()

        kfix = jnp.float32(2.0 ** 99)

        def _dec(w):
            outs = []
            for dd in range(4):
                t = w >> (24 - 8 * dd) if dd < 3 else w
                b = ((t & 0x7F) << 21) | ((t & 0x80) << 24)
                outs.append(lax.bitcast_convert_type(b, jnp.float32))
            return outs

        def phase_c(i, c2):
            o16 = i * 16
            for l in range(L):
                fx = frb[gb, pl.ds(l * C + o16, 16)]
                fy = frb[gb, pl.ds((L + l) * C + o16, 16)]
                fz = frb[gb, pl.ds((2 * L + l) * C + o16, 16)]
                m = []
                for cyz in range(4):
                    da = _dec(rpk[gb, l * 8 + 2 * cyz, pl.ds(o16, 16)])
                    db = _dec(rpk[gb, l * 8 + 2 * cyz + 1, pl.ds(o16, 16)])
                    m.append(tuple(da[d] + (db[d] - da[d]) * fx for d in range(4)))
                n0 = tuple(m[0][d] + (m[1][d] - m[0][d]) * fy for d in range(4))
                n1 = tuple(m[2][d] + (m[3][d] - m[2][d]) * fy for d in range(4))
                for d in range(4):
                    accb[l * 4 + d, pl.ds(o16, 16)] = (n0[d] + (n1[d] - n0[d]) * fz) * kfix
            return c2

        lax.fori_loop(0, C // 16, phase_c, 0)

        pltpu.sync_copy(accb, out.at[:, pl.ds(base + p0, C)])
        return carry

    lax.fori_loop(0, nch, chunk_body, 0)


def _sc_encode(x, tpk):
    n = x.shape[0]
    npts = n // NW
    total = tpk.shape[0]
    mesh = plsc.VectorSubcoreMesh(core_axis_name="c", subcore_axis_name="s")
    f = pl.kernel(
        _enc_body,
        out_type=jax.ShapeDtypeStruct((L * DIM, n), jnp.float32),
        mesh=mesh,
        compiler_params=pltpu.CompilerParams(
            use_tc_tiling_on_sc=False, needs_layout_passes=False),
        scratch_types=[
            pltpu.VMEM_SHARED((total,), jnp.int32),
            pltpu.VMEM((2, 3, C), jnp.float32),
            pltpu.VMEM((2, L * 8, C), jnp.int32),
            pltpu.VMEM((2, 3 * L * C), jnp.float32),
            pltpu.VMEM((2, L * 8, C), jnp.int32),
            pltpu.VMEM((L * DIM, C), jnp.float32),
            pltpu.SemaphoreType.DMA,
            pltpu.SemaphoreType.DMA,
        ],
    )
    return f(tpk, x[:, 0], x[:, 1], x[:, 2])


def _pack_table(table):
    tb = lax.bitcast_convert_type((table * 8192.0).astype(jnp.float8_e5m2),
                                  jnp.uint8).astype(jnp.uint32)
    w = (tb[:, 0] << 24) | (tb[:, 1] << 16) | (tb[:, 2] << 8) | tb[:, 3]
    pad = (-table.shape[0]) % (16 * 8)
    return jnp.pad(w, (0, pad)).astype(jnp.int32)


def _f16_round(v):
    b = jax.lax.bitcast_convert_type(v, jnp.int32)
    b = (b + jnp.int32(0xFFF) + ((b >> 13) & 1)) & jnp.int32(-8192)
    return jax.lax.bitcast_convert_type(b, jnp.float32)


def _selu(v):
    return _SELU_SCALE * jnp.where(v > 0, v, _SELU_ALPHA * (jnp.exp(v) - 1.0))


def _head_body(enc_ref, cr_ref, w1_ref, b1_ref, w2_ref, b2_ref, w3_ref, b3_ref, out_ref):
    enc = enc_ref[...]
    cr = cr_ref[...]
    cr_h = _f16_round(cr) * SCALE_MULTI
    n_mod = (jnp.arange(L * DIM, dtype=jnp.int32) % L).reshape(L * DIM, 1).astype(jnp.float32)
    denom = jnp.sqrt(jnp.maximum(PLS * 4.0 * n_mod * cr_h, 1e-8))
    scaling = _f16_round(jax.lax.erf(1.0 / jnp.maximum(denom, 1e-8)))
    h = enc * scaling
    h1 = _selu(jnp.dot(w1_ref[...], h, preferred_element_type=jnp.float32) + b1_ref[...])
    h2 = _selu(jnp.dot(w2_ref[...], h1, preferred_element_type=jnp.float32) + b2_ref[...])
    sigma = jnp.dot(w3_ref[...], h2, preferred_element_type=jnp.float32) + b3_ref[...]
    out_ref[...] = jnp.exp(jnp.clip(sigma + DENSITY_OFFSET, -15.0, 15.0))


def _mlp_head(enc_t, cr, W1, b1, W2, b2, W3, b3, blk=2048):
    n = enc_t.shape[1]
    grid = n // blk
    full = lambda *shape: pl.BlockSpec(shape, lambda i: (0,) * len(shape))
    out = pl.pallas_call(
        _head_body,
        grid=(grid,),
        in_specs=[
            pl.BlockSpec((L * DIM, blk), lambda i: (0, i)),
            pl.BlockSpec((1, blk), lambda i: (0, i)),
            full(HIDDEN, L * DIM),
            full(HIDDEN, 1),
            full(HIDDEN, HIDDEN),
            full(HIDDEN, 1),
            full(1, HIDDEN),
            full(1, 1),
        ],
        out_specs=pl.BlockSpec((1, blk), lambda i: (0, i)),
        out_shape=jax.ShapeDtypeStruct((1, n), jnp.float32),
    )(enc_t, cr.reshape(1, n), W1.T, b1.reshape(HIDDEN, 1),
      W2.T, b2.reshape(HIDDEN, 1), W3.T, b3.reshape(1, 1))
    return out.reshape(n, 1)


def kernel(x, cr, table, W1, b1, W2, b2, W3, b3):
    tpk = _pack_table(table)
    n = x.shape[0]
    h = n // 2
    enc0 = _sc_encode(x[:h], tpk)
    enc1 = _sc_encode(x[h:], tpk)
    d0 = _mlp_head(enc0, cr[:h], W1, b1, W2, b2, W3, b3)
    d1 = _mlp_head(enc1, cr[h:], W1, b1, W2, b2, W3, b3)
    return jnp.concatenate([d0, d1], axis=0)


# speedup vs baseline: 1.5376x; 1.2430x over previous
import functools

import jax
import jax.numpy as jnp
import numpy as np
from jax import lax
from jax.experimental import pallas as pl
from jax.experimental.pallas import tpu as pltpu
from jax.experimental.pallas import tpu_sc as plsc

SCALE_MULTI = 0.5
LOG2_T = 16
BASE_RES = 16
PLS = 2
L = 10
DIM = 4
HIDDEN = 64
DENSITY_OFFSET = -4.0

_P2 = int(np.uint32(2654435761).view(np.int32))
_P3 = int(np.uint32(805459861).view(np.int32))

_SELU_ALPHA = 1.6732632423543772
_SELU_SCALE = 1.0507009873554805

NW = 32
C = 128


def _levels():
    T = 2 ** LOG2_T
    out = []
    off = 0
    for l in range(L):
        res = int(np.floor(BASE_RES * (PLS ** l)))
        size = min((res + 1) ** 3, T)
        out.append((res, size, off, size == (res + 1) ** 3))
        off += size
    return out

_LEVELS = _levels()
_TOTAL = sum(s for _, s, _, _ in _LEVELS)


def _enc_body(tpk, xx, xy, xz, out,
              spk, xcb, idxb, frb, rpk, accb, sem, semx):
    npts = out.shape[1] // NW
    nch = npts // C
    cid = lax.axis_index("c")
    sid = lax.axis_index("s")
    wid = sid * 2 + cid
    base = wid * npts

    total = tpk.shape[0]
    share = total // 16
    pltpu.sync_copy(tpk.at[pl.ds(sid * share, share)], spk.at[pl.ds(sid * share, share)])
    plsc.subcore_barrier()

    def xc_copies(g, buf):
        p0 = g * C
        return [pltpu.make_async_copy(xx.at[pl.ds(base + p0, C)], xcb.at[buf, 0], semx),
                pltpu.make_async_copy(xy.at[pl.ds(base + p0, C)], xcb.at[buf, 1], semx),
                pltpu.make_async_copy(xz.at[pl.ds(base + p0, C)], xcb.at[buf, 2], semx)]

    def a_and_fire(k):
        kb = k & 1
        for cp in xc_copies(k, kb):
            cp.wait()

        @pl.when(k + 1 < nch)
        def _prefetch():
            for cp in xc_copies(k + 1, 1 - kb):
                cp.start()

        def phase_a(i, c2):
            off = i * 16
            vx = xcb[kb, 0, pl.ds(off, 16)]
            vy = xcb[kb, 1, pl.ds(off, 16)]
            vz = xcb[kb, 2, pl.ds(off, 16)]
            for l, (res, size, loff, dense) in enumerate(_LEVELS):
                px = vx * float(res)
                py = vy * float(res)
                pz = vz * float(res)
                ix = px.astype(jnp.int32)
                iy = py.astype(jnp.int32)
                iz = pz.astype(jnp.int32)
                frb[kb, pl.ds(l * C + i * 16, 16)] = px - ix.astype(jnp.float32)
                frb[kb, pl.ds((L + l) * C + i * 16, 16)] = py - iy.astype(jnp.float32)
                frb[kb, pl.ds((2 * L + l) * C + i * 16, 16)] = pz - iz.astype(jnp.float32)
                if dense:
                    r1 = res + 1
                    ax = (ix, ix + 1)
                    ay = (iy * r1, iy * r1 + r1)
                    az = (iz * (r1 * r1) + loff, iz * (r1 * r1) + (r1 * r1) + loff)
                    for c in range(8):
                        v = ax[c & 1] + ay[(c >> 1) & 1] + az[(c >> 2) & 1]
                        idxb[kb, pl.ds((l * 8 + c) * C + i * 16, 16)] = v
                else:
                    hx = (ix, ix + 1)
                    hy = (iy * _P2, iy * _P2 + _P2)
                    hz = (iz * _P3, iz * _P3 + _P3)
                    for c in range(8):
                        h = hx[c & 1] ^ hy[(c >> 1) & 1] ^ hz[(c >> 2) & 1]
                        idxb[kb, pl.ds((l * 8 + c) * C + i * 16, 16)] = (h & 0xFFFF) + loff
            return c2

        lax.fori_loop(0, C // 16, phase_a, 0)

        pltpu.make_async_copy(spk.at[idxb.at[kb]], rpk.at[kb], sem).start()

    for cp in xc_copies(0, 0):
        cp.start()
    a_and_fire(0)

    def chunk_body(g, carry):
        p0 = g * C
        gb = g & 1

        @pl.when(g + 1 < nch)
        def _next():
            a_and_fire(g + 1)

        pltpu.make_async_copy(spk.at[idxb.at[gb]], rpk.at[gb], sem).wait()

        kfix = jnp.float32(2.0 ** 99)

        def _dec(w):
            outs = []
            for dd in range(4):
                t = w >> (24 - 8 * dd) if dd < 3 else w
                b = ((t & 0x7F) << 21) | ((t & 0x80) << 24)
                outs.append(lax.bitcast_convert_type(b, jnp.float32))
            return outs

        def phase_c(i, c2):
            o16 = i * 16
            for l in range(L):
                fx = frb[gb, pl.ds(l * C + o16, 16)]
                fy = frb[gb, pl.ds((L + l) * C + o16, 16)]
                fz = frb[gb, pl.ds((2 * L + l) * C + o16, 16)]
                m = []
                for cyz in range(4):
                    da = _dec(rpk[gb, pl.ds((l * 8 + 2 * cyz) * C + o16, 16)])
                    db = _dec(rpk[gb, pl.ds((l * 8 + 2 * cyz + 1) * C + o16, 16)])
                    m.append(tuple(da[d] + (db[d] - da[d]) * fx for d in range(4)))
                n0 = tuple(m[0][d] + (m[1][d] - m[0][d]) * fy for d in range(4))
                n1 = tuple(m[2][d] + (m[3][d] - m[2][d]) * fy for d in range(4))
                for d in range(4):
                    accb[l * 4 + d, pl.ds(o16, 16)] = (n0[d] + (n1[d] - n0[d]) * fz) * kfix
            return c2

        lax.fori_loop(0, C // 16, phase_c, 0)

        pltpu.sync_copy(accb, out.at[:, pl.ds(base + p0, C)])
        return carry

    lax.fori_loop(0, nch, chunk_body, 0)


def _sc_encode(x, tpk):
    n = x.shape[0]
    npts = n // NW
    total = tpk.shape[0]
    mesh = plsc.VectorSubcoreMesh(core_axis_name="c", subcore_axis_name="s")
    f = pl.kernel(
        _enc_body,
        out_type=jax.ShapeDtypeStruct((L * DIM, n), jnp.float32),
        mesh=mesh,
        compiler_params=pltpu.CompilerParams(
            use_tc_tiling_on_sc=False, needs_layout_passes=False),
        scratch_types=[
            pltpu.VMEM_SHARED((total,), jnp.int32),
            pltpu.VMEM((2, 3, C), jnp.float32),
            pltpu.VMEM((2, L * 8 * C), jnp.int32),
            pltpu.VMEM((2, 3 * L * C), jnp.float32),
            pltpu.VMEM((2, L * 8 * C), jnp.int32),
            pltpu.VMEM((L * DIM, C), jnp.float32),
            pltpu.SemaphoreType.DMA,
            pltpu.SemaphoreType.DMA,
        ],
    )
    return f(tpk, x[:, 0], x[:, 1], x[:, 2])


def _pack_table(table):
    tb = lax.bitcast_convert_type((table * 8192.0).astype(jnp.float8_e5m2),
                                  jnp.uint8).astype(jnp.uint32)
    w = (tb[:, 0] << 24) | (tb[:, 1] << 16) | (tb[:, 2] << 8) | tb[:, 3]
    pad = (-table.shape[0]) % (16 * 8)
    return jnp.pad(w, (0, pad)).astype(jnp.int32)


def _f16_round(v):
    b = jax.lax.bitcast_convert_type(v, jnp.int32)
    b = (b + jnp.int32(0xFFF) + ((b >> 13) & 1)) & jnp.int32(-8192)
    return jax.lax.bitcast_convert_type(b, jnp.float32)


def _selu(v):
    return _SELU_SCALE * jnp.where(v > 0, v, _SELU_ALPHA * (jnp.exp(v) - 1.0))


def _head_body(enc_ref, cr_ref, w1_ref, b1_ref, w2_ref, b2_ref, w3_ref, b3_ref, out_ref):
    enc = enc_ref[...]
    cr = cr_ref[...]
    cr_h = _f16_round(cr) * SCALE_MULTI
    n_mod = (jnp.arange(L * DIM, dtype=jnp.int32) % L).reshape(L * DIM, 1).astype(jnp.float32)
    denom = jnp.sqrt(jnp.maximum(PLS * 4.0 * n_mod * cr_h, 1e-8))
    scaling = _f16_round(jax.lax.erf(1.0 / jnp.maximum(denom, 1e-8)))
    h = enc * scaling
    h1 = _selu(jnp.dot(w1_ref[...], h, preferred_element_type=jnp.float32) + b1_ref[...])
    h2 = _selu(jnp.dot(w2_ref[...], h1, preferred_element_type=jnp.float32) + b2_ref[...])
    sigma = jnp.dot(w3_ref[...], h2, preferred_element_type=jnp.float32) + b3_ref[...]
    out_ref[...] = jnp.exp(jnp.clip(sigma + DENSITY_OFFSET, -15.0, 15.0))


def _mlp_head(enc_t, cr, W1, b1, W2, b2, W3, b3, blk=2048):
    n = enc_t.shape[1]
    grid = n // blk
    full = lambda *shape: pl.BlockSpec(shape, lambda i: (0,) * len(shape))
    out = pl.pallas_call(
        _head_body,
        grid=(grid,),
        in_specs=[
            pl.BlockSpec((L * DIM, blk), lambda i: (0, i)),
            pl.BlockSpec((1, blk), lambda i: (0, i)),
            full(HIDDEN, L * DIM),
            full(HIDDEN, 1),
            full(HIDDEN, HIDDEN),
            full(HIDDEN, 1),
            full(1, HIDDEN),
            full(1, 1),
        ],
        out_specs=pl.BlockSpec((1, blk), lambda i: (0, i)),
        out_shape=jax.ShapeDtypeStruct((1, n), jnp.float32),
    )(enc_t, cr.reshape(1, n), W1.T, b1.reshape(HIDDEN, 1),
      W2.T, b2.reshape(HIDDEN, 1), W3.T, b3.reshape(1, 1))
    return out.reshape(n, 1)


def kernel(x, cr, table, W1, b1, W2, b2, W3, b3):
    tpk = _pack_table(table)
    enc_t = _sc_encode(x, tpk)
    return _mlp_head(enc_t, cr, W1, b1, W2, b2, W3, b3)
